# Initial kernel scaffold; baseline (speedup 1.0000x reference)
import jax
import jax.numpy as jnp
from jax.experimental import pallas as pl


def kernel(inputs, update_mask, apply_mask):
    raise NotImplementedError("write your pallas kernel here")


# speedup vs baseline: 24.8726x; 24.8726x over previous
import functools

import jax
import jax.numpy as jnp
from jax import lax
from jax.experimental import pallas as pl
from jax.experimental.pallas import tpu as pltpu
from jax.experimental.pallas import tpu_sc as plsc

_N_ROWS, _N_COLS = 4096, 8192
_TOTAL = _N_ROWS * _N_COLS
_NC, _NS = 2, 16
_NW = _NC * _NS
_PER_W = _TOTAL // _NW
_CHUNK = 16384
_N_CHUNKS = _PER_W // _CHUNK
_VECS = _CHUNK // 16


def _make_sc_prune():
    mesh = plsc.VectorSubcoreMesh(
        core_axis_name="c", subcore_axis_name="s",
        num_cores=_NC, num_subcores=_NS)

    @functools.partial(
        pl.kernel,
        out_type=jax.ShapeDtypeStruct((_TOTAL,), jnp.float32),
        mesh=mesh,
        scratch_types=[
            pltpu.VMEM((_CHUNK,), jnp.float32),
            pltpu.VMEM((_CHUNK,), jnp.float32),
            pltpu.VMEM((16,), jnp.int32),
        ],
    )
    def sc_prune(x_hbm, flag_hbm, out_hbm, in_v, out_v, flag_v):
        wid = lax.axis_index("s") * _NC + lax.axis_index("c")
        base = wid * _PER_W

        lanes = lax.iota(jnp.int32, 16)
        o = lanes & 3
        g4 = lanes - o
        p1 = g4 + ((o + 1) & 3)
        p2 = g4 + ((o + 2) & 3)
        p3 = g4 + ((o + 3) & 3)
        m1 = o == 3
        m2 = o >= 2
        m3 = o >= 1

        pltpu.sync_copy(flag_hbm, flag_v)
        flag = flag_v[...] != 0

        def chunk_body(ci, _):
            off = base + ci * _CHUNK
            pltpu.sync_copy(x_hbm.at[pl.ds(off, _CHUNK)], in_v)

            def vec_body(vi, _):
                s = vi * 16
                v = in_v[pl.ds(s, 16)]
                a = jnp.abs(v)
                q1 = a.at[p1].get(mode="promise_in_bounds")
                q2 = a.at[p2].get(mode="promise_in_bounds")
                q3 = a.at[p3].get(mode="promise_in_bounds")
                l1 = (q1 > a) | (m1 & (q1 == a))
                l2 = (q2 > a) | (m2 & (q2 == a))
                l3 = (q3 > a) | (m3 & (q3 == a))
                drop = (l1 & l2) | (l1 & l3) | (l2 & l3)
                drop = drop & flag
                out_v[pl.ds(s, 16)] = jnp.where(drop, jnp.float32(0), v)
                return ()

            lax.fori_loop(0, _VECS, vec_body, ())
            pltpu.sync_copy(out_v, out_hbm.at[pl.ds(off, _CHUNK)])
            return ()

        lax.fori_loop(0, _N_CHUNKS, chunk_body, ())

    return sc_prune


_sc_prune = _make_sc_prune()


def kernel(inputs, update_mask, apply_mask):
    flat = inputs.reshape(_TOTAL)
    flag = (update_mask != 0) & (apply_mask != 0)
    flag_v = jnp.broadcast_to(jnp.asarray(flag, jnp.int32), (16,))
    out = _sc_prune(flat, flag_v)
    return out.reshape(inputs.shape)


# speedup vs baseline: 30.6717x; 1.2332x over previous
import functools

import jax
import jax.numpy as jnp
from jax import lax
from jax.experimental import pallas as pl
from jax.experimental.pallas import tpu as pltpu
from jax.experimental.pallas import tpu_sc as plsc

_N_ROWS, _N_COLS = 4096, 8192
_TOTAL = _N_ROWS * _N_COLS
_NC, _NS = 2, 16
_NW = _NC * _NS
_PER_W = _TOTAL // _NW
_CHUNK = 16384
_N_CHUNKS = _PER_W // _CHUNK
_N_ITERS = _N_CHUNKS // 2
_VECS = _CHUNK // 16


def _make_sc_prune():
    mesh = plsc.VectorSubcoreMesh(
        core_axis_name="c", subcore_axis_name="s",
        num_cores=_NC, num_subcores=_NS)

    @functools.partial(
        pl.kernel,
        out_type=jax.ShapeDtypeStruct((_TOTAL,), jnp.float32),
        mesh=mesh,
        scratch_types=[
            pltpu.VMEM((_CHUNK,), jnp.float32),
            pltpu.VMEM((_CHUNK,), jnp.float32),
            pltpu.VMEM((_CHUNK,), jnp.float32),
            pltpu.VMEM((_CHUNK,), jnp.float32),
            pltpu.VMEM((16,), jnp.int32),
            pltpu.SemaphoreType.DMA,
            pltpu.SemaphoreType.DMA,
            pltpu.SemaphoreType.DMA,
            pltpu.SemaphoreType.DMA,
        ],
    )
    def sc_prune(x_hbm, flag_hbm, out_hbm, in0, in1, out0, out1, flag_v,
                 si0, si1, so0, so1):
        wid = lax.axis_index("s") * _NC + lax.axis_index("c")
        base = wid * _PER_W

        lanes = lax.iota(jnp.int32, 16)
        o = lanes & 3
        g4 = lanes - o
        p1 = g4 + ((o + 1) & 3)
        p2 = g4 + ((o + 2) & 3)
        p3 = g4 + ((o + 3) & 3)
        m1 = o == 3
        m2 = o >= 2
        m3 = o >= 1

        pltpu.sync_copy(flag_hbm, flag_v)
        flag = flag_v[...] != 0

        def compute(in_v, out_v):
            def _vec(vi, _):
                s = vi * 16
                v = in_v[pl.ds(s, 16)]
                a = jnp.abs(v)
                q1 = a.at[p1].get(mode="promise_in_bounds")
                q2 = a.at[p2].get(mode="promise_in_bounds")
                q3 = a.at[p3].get(mode="promise_in_bounds")
                l1 = (q1 > a) | (m1 & (q1 == a))
                l2 = (q2 > a) | (m2 & (q2 == a))
                l3 = (q3 > a) | (m3 & (q3 == a))
                drop = (l1 & l2) | (l1 & l3) | (l2 & l3)
                drop = drop & flag
                out_v[pl.ds(s, 16)] = jnp.where(drop, jnp.float32(0), v)
                return ()

            lax.fori_loop(0, _VECS, _vec, ())

        def in_slice(c):
            return x_hbm.at[pl.ds(base + c * _CHUNK, _CHUNK)]

        def out_slice(c):
            return out_hbm.at[pl.ds(base + c * _CHUNK, _CHUNK)]

        pltpu.async_copy(in_slice(0), in0, si0)
        pltpu.async_copy(in_slice(1), in1, si1)

        def iter_body(it, _):
            c0 = it * 2
            c1 = c0 + 1

            pltpu.make_async_copy(in_slice(c0), in0, si0).wait()

            @pl.when(it > 0)
            def _():
                pltpu.make_async_copy(out0, out_slice(c0), so0).wait()

            compute(in0, out0)
            pltpu.async_copy(out0, out_slice(c0), so0)

            @pl.when(it < _N_ITERS - 1)
            def _():
                pltpu.async_copy(in_slice(c0 + 2), in0, si0)

            pltpu.make_async_copy(in_slice(c1), in1, si1).wait()

            @pl.when(it > 0)
            def _():
                pltpu.make_async_copy(out1, out_slice(c1), so1).wait()

            compute(in1, out1)
            pltpu.async_copy(out1, out_slice(c1), so1)

            @pl.when(it < _N_ITERS - 1)
            def _():
                pltpu.async_copy(in_slice(c1 + 2), in1, si1)

            return ()

        lax.fori_loop(0, _N_ITERS, iter_body, ())
        pltpu.make_async_copy(out0, out_slice(_N_CHUNKS - 2), so0).wait()
        pltpu.make_async_copy(out1, out_slice(_N_CHUNKS - 1), so1).wait()

    return sc_prune


_sc_prune = _make_sc_prune()


def kernel(inputs, update_mask, apply_mask):
    flat = inputs.reshape(_TOTAL)
    flag = (update_mask != 0) & (apply_mask != 0)
    flag_v = jnp.broadcast_to(jnp.asarray(flag, jnp.int32), (16,))
    out = _sc_prune(flat, flag_v)
    return out.reshape(inputs.shape)


# speedup vs baseline: 38.3961x; 1.2518x over previous
import functools

import jax
import jax.numpy as jnp
from jax import lax
from jax.experimental import pallas as pl
from jax.experimental.pallas import tpu as pltpu
from jax.experimental.pallas import tpu_sc as plsc

_N_ROWS, _N_COLS = 4096, 8192
_TOTAL = _N_ROWS * _N_COLS
_NC, _NS = 2, 16
_NW = _NC * _NS
_PER_W = _TOTAL // _NW
_CHUNK = 16384
_N_CHUNKS = _PER_W // _CHUNK
_N_ITERS = _N_CHUNKS // 2
_VECS = _CHUNK // 16


def _make_sc_prune():
    mesh = plsc.VectorSubcoreMesh(
        core_axis_name="c", subcore_axis_name="s",
        num_cores=_NC, num_subcores=_NS)

    @functools.partial(
        pl.kernel,
        out_type=jax.ShapeDtypeStruct((_TOTAL,), jnp.float32),
        mesh=mesh,
        scratch_types=[
            pltpu.VMEM((_CHUNK,), jnp.float32),
            pltpu.VMEM((_CHUNK,), jnp.float32),
            pltpu.VMEM((_CHUNK,), jnp.float32),
            pltpu.VMEM((_CHUNK,), jnp.float32),
            pltpu.SemaphoreType.DMA,
            pltpu.SemaphoreType.DMA,
            pltpu.SemaphoreType.DMA,
            pltpu.SemaphoreType.DMA,
        ],
    )
    def sc_prune(x_hbm, out_hbm, in0, in1, out0, out1,
                 si0, si1, so0, so1):
        wid = lax.axis_index("s") * _NC + lax.axis_index("c")
        base = wid * _PER_W

        lanes = lax.iota(jnp.int32, 16)
        o = lanes & 3
        g4 = lanes - o
        p1 = g4 + ((o + 1) & 3)
        p2 = g4 + ((o + 2) & 3)
        p3 = g4 + ((o + 3) & 3)
        md1 = (o + 1) >> 2
        md2 = (o >> 1) & 1
        md3 = (o + 3) >> 2

        def compute(in_v, out_v):
            @plsc.parallel_loop(0, _VECS, 1, unroll=4)
            def _vec(vi):
                s = vi * 16
                v = in_v[pl.ds(s, 16)]
                a = jnp.abs(v)
                abits = lax.bitcast_convert_type(a, jnp.int32)
                q1 = lax.bitcast_convert_type(a.at[p1].get(mode="promise_in_bounds"), jnp.int32)
                q2 = lax.bitcast_convert_type(a.at[p2].get(mode="promise_in_bounds"), jnp.int32)
                q3 = lax.bitcast_convert_type(a.at[p3].get(mode="promise_in_bounds"), jnp.int32)
                l1 = q1 > abits - md1
                l2 = q2 > abits - md2
                l3 = q3 > abits - md3
                drop = (l1 & l2) | (l3 & (l1 | l2))
                out_v[pl.ds(s, 16)] = jnp.where(drop, jnp.float32(0), v)

        def in_slice(c):
            return x_hbm.at[pl.ds(base + c * _CHUNK, _CHUNK)]

        def out_slice(c):
            return out_hbm.at[pl.ds(base + c * _CHUNK, _CHUNK)]

        pltpu.async_copy(in_slice(0), in0, si0)
        pltpu.async_copy(in_slice(1), in1, si1)

        def iter_body(it, _):
            c0 = it * 2
            c1 = c0 + 1

            pltpu.make_async_copy(in_slice(c0), in0, si0).wait()

            @pl.when(it > 0)
            def _():
                pltpu.make_async_copy(out0, out_slice(c0), so0).wait()

            compute(in0, out0)
            pltpu.async_copy(out0, out_slice(c0), so0)

            @pl.when(it < _N_ITERS - 1)
            def _():
                pltpu.async_copy(in_slice(c0 + 2), in0, si0)

            pltpu.make_async_copy(in_slice(c1), in1, si1).wait()

            @pl.when(it > 0)
            def _():
                pltpu.make_async_copy(out1, out_slice(c1), so1).wait()

            compute(in1, out1)
            pltpu.async_copy(out1, out_slice(c1), so1)

            @pl.when(it < _N_ITERS - 1)
            def _():
                pltpu.async_copy(in_slice(c1 + 2), in1, si1)

            return ()

        lax.fori_loop(0, _N_ITERS, iter_body, ())
        pltpu.make_async_copy(out0, out_slice(_N_CHUNKS - 2), so0).wait()
        pltpu.make_async_copy(out1, out_slice(_N_CHUNKS - 1), so1).wait()

    return sc_prune


_sc_prune = _make_sc_prune()


def kernel(inputs, update_mask, apply_mask):
    flat = inputs.reshape(_TOTAL)
    flag = (update_mask != 0) & (apply_mask != 0)
    out = lax.cond(flag, _sc_prune, lambda x: x, flat)
    return out.reshape(inputs.shape)


# speedup vs baseline: 45.7704x; 1.1921x over previous
import functools

import jax
import jax.numpy as jnp
from jax import lax
from jax.experimental import pallas as pl
from jax.experimental.pallas import tpu as pltpu
from jax.experimental.pallas import tpu_sc as plsc

_N_ROWS, _N_COLS = 4096, 8192
_NC, _NS = 2, 16
_NW = _NC * _NS
_ROWS_W = _N_ROWS // _NW
_BR, _BC = 8, 2048
_CB = _N_COLS // _BC
_N_CHUNKS = (_ROWS_W // _BR) * _CB
_N_ITERS = _N_CHUNKS // 2
_VECS = (_BR * _BC) // 16
_VPR = _BC // 16


def _make_sc_prune():
    mesh = plsc.VectorSubcoreMesh(
        core_axis_name="c", subcore_axis_name="s",
        num_cores=_NC, num_subcores=_NS)

    @functools.partial(
        pl.kernel,
        out_type=jax.ShapeDtypeStruct((_N_ROWS, _N_COLS), jnp.float32),
        mesh=mesh,
        compiler_params=pltpu.CompilerParams(use_tc_tiling_on_sc=True),
        scratch_types=[
            pltpu.VMEM((_BR, _BC), jnp.float32),
            pltpu.VMEM((_BR, _BC), jnp.float32),
            pltpu.VMEM((_BR, _BC), jnp.float32),
            pltpu.VMEM((_BR, _BC), jnp.float32),
            pltpu.SemaphoreType.DMA,
            pltpu.SemaphoreType.DMA,
            pltpu.SemaphoreType.DMA,
            pltpu.SemaphoreType.DMA,
        ],
    )
    def sc_prune(x_hbm, out_hbm, in0, in1, out0, out1, si0, si1, so0, so1):
        wid = lax.axis_index("s") * _NC + lax.axis_index("c")
        base_row = wid * _ROWS_W

        lanes = lax.iota(jnp.int32, 16)
        o = lanes & 3
        g4 = lanes - o
        p1 = g4 + ((o + 1) & 3)
        p2 = g4 + ((o + 2) & 3)
        p3 = g4 + ((o + 3) & 3)
        md1 = (o + 1) >> 2
        md2 = (o >> 1) & 1
        md3 = (o + 3) >> 2

        def compute(in_v, out_v):
            @plsc.parallel_loop(0, _VECS, 1, unroll=4)
            def _vec(vi):
                ri = vi >> 7
                s = (vi & (_VPR - 1)) * 16
                v = in_v[ri, pl.ds(s, 16)]
                a = jnp.abs(v)
                abits = lax.bitcast_convert_type(a, jnp.int32)
                q1 = lax.bitcast_convert_type(
                    a.at[p1].get(mode="promise_in_bounds"), jnp.int32)
                q2 = lax.bitcast_convert_type(
                    a.at[p2].get(mode="promise_in_bounds"), jnp.int32)
                q3 = lax.bitcast_convert_type(
                    a.at[p3].get(mode="promise_in_bounds"), jnp.int32)
                l1 = q1 > abits - md1
                l2 = q2 > abits - md2
                l3 = q3 > abits - md3
                drop = (l1 & l2) | (l3 & (l1 | l2))
                out_v[ri, pl.ds(s, 16)] = jnp.where(drop, jnp.float32(0), v)

        def in_slice(c):
            r0 = base_row + (c >> 2) * _BR
            c0 = (c & (_CB - 1)) * _BC
            return x_hbm.at[pl.ds(r0, _BR), pl.ds(c0, _BC)]

        def out_slice(c):
            r0 = base_row + (c >> 2) * _BR
            c0 = (c & (_CB - 1)) * _BC
            return out_hbm.at[pl.ds(r0, _BR), pl.ds(c0, _BC)]

        pltpu.async_copy(in_slice(0), in0, si0)
        pltpu.async_copy(in_slice(1), in1, si1)

        def iter_body(it, _):
            c0 = it * 2
            c1 = c0 + 1

            pltpu.make_async_copy(in_slice(c0), in0, si0).wait()

            @pl.when(it > 0)
            def _():
                pltpu.make_async_copy(out0, out_slice(c0), so0).wait()

            compute(in0, out0)
            pltpu.async_copy(out0, out_slice(c0), so0)

            @pl.when(it < _N_ITERS - 1)
            def _():
                pltpu.async_copy(in_slice(c0 + 2), in0, si0)

            pltpu.make_async_copy(in_slice(c1), in1, si1).wait()

            @pl.when(it > 0)
            def _():
                pltpu.make_async_copy(out1, out_slice(c1), so1).wait()

            compute(in1, out1)
            pltpu.async_copy(out1, out_slice(c1), so1)

            @pl.when(it < _N_ITERS - 1)
            def _():
                pltpu.async_copy(in_slice(c1 + 2), in1, si1)

            return ()

        lax.fori_loop(0, _N_ITERS, iter_body, ())
        pltpu.make_async_copy(out0, out_slice(_N_CHUNKS - 2), so0).wait()
        pltpu.make_async_copy(out1, out_slice(_N_CHUNKS - 1), so1).wait()

    return sc_prune


_sc_prune = _make_sc_prune()


def kernel(inputs, update_mask, apply_mask):
    flag = (update_mask != 0) & (apply_mask != 0)
    return lax.cond(flag, _sc_prune, lambda x: x, inputs)


# speedup vs baseline: 74.3794x; 1.6251x over previous
import functools

import jax
import jax.numpy as jnp
from jax import lax
from jax.experimental import pallas as pl
from jax.experimental.pallas import tpu as pltpu
from jax.experimental.pallas import tpu_sc as plsc

_N_ROWS, _N_COLS = 4096, 8192
_NC, _NS = 2, 16
_NW = _NC * _NS
_ROWS_W = _N_ROWS // _NW
_BR, _BC = 8, 2048
_CB = _N_COLS // _BC
_N_CHUNKS = (_ROWS_W // _BR) * _CB
_N_ITERS = _N_CHUNKS // 2
_VECS = (_BR * _BC) // 16
_VPR = _BC // 16


def _make_sc_prune():
    mesh = plsc.VectorSubcoreMesh(
        core_axis_name="c", subcore_axis_name="s",
        num_cores=_NC, num_subcores=_NS)

    @functools.partial(
        pl.kernel,
        out_type=jax.ShapeDtypeStruct((_N_ROWS, _N_COLS), jnp.float32),
        mesh=mesh,
        compiler_params=pltpu.CompilerParams(use_tc_tiling_on_sc=True),
        scratch_types=[
            pltpu.VMEM((_BR, _BC), jnp.float32),
            pltpu.VMEM((_BR, _BC), jnp.float32),
            pltpu.VMEM((_BR, _BC), jnp.float32),
            pltpu.VMEM((_BR, _BC), jnp.float32),
            pltpu.VMEM((16,), jnp.int32),
            pltpu.SemaphoreType.DMA,
            pltpu.SemaphoreType.DMA,
            pltpu.SemaphoreType.DMA,
            pltpu.SemaphoreType.DMA,
        ],
    )
    def sc_prune(x_hbm, flag_hbm, out_hbm, in0, in1, out0, out1, flag_v,
                 si0, si1, so0, so1):
        wid = lax.axis_index("s") * _NC + lax.axis_index("c")
        base_row = wid * _ROWS_W

        lanes = lax.iota(jnp.int32, 16)
        o = lanes & 3
        g4 = lanes - o
        p1 = g4 + ((o + 1) & 3)
        p2 = g4 + ((o + 2) & 3)
        p3 = g4 + ((o + 3) & 3)
        md1 = (o + 1) >> 2
        md2 = (o >> 1) & 1
        md3 = (o + 3) >> 2

        pltpu.sync_copy(flag_hbm, flag_v)
        flag = flag_v[...] != 0

        def compute(in_v, out_v):
            @plsc.parallel_loop(0, _VECS, 1, unroll=4)
            def _vec(vi):
                ri = vi >> 7
                s = (vi & (_VPR - 1)) * 16
                v = in_v[ri, pl.ds(s, 16)]
                a = jnp.abs(v)
                abits = lax.bitcast_convert_type(a, jnp.int32)
                q1 = lax.bitcast_convert_type(
                    a.at[p1].get(mode="promise_in_bounds"), jnp.int32)
                q2 = lax.bitcast_convert_type(
                    a.at[p2].get(mode="promise_in_bounds"), jnp.int32)
                q3 = lax.bitcast_convert_type(
                    a.at[p3].get(mode="promise_in_bounds"), jnp.int32)
                l1 = q1 > abits - md1
                l2 = q2 > abits - md2
                l3 = q3 > abits - md3
                drop = ((l1 & l2) | (l3 & (l1 | l2))) & flag
                out_v[ri, pl.ds(s, 16)] = jnp.where(drop, jnp.float32(0), v)

        def in_slice(c):
            r0 = base_row + (c >> 2) * _BR
            c0 = (c & (_CB - 1)) * _BC
            return x_hbm.at[pl.ds(r0, _BR), pl.ds(c0, _BC)]

        def out_slice(c):
            r0 = base_row + (c >> 2) * _BR
            c0 = (c & (_CB - 1)) * _BC
            return out_hbm.at[pl.ds(r0, _BR), pl.ds(c0, _BC)]

        pltpu.async_copy(in_slice(0), in0, si0)
        pltpu.async_copy(in_slice(1), in1, si1)

        def iter_body(it, _):
            c0 = it * 2
            c1 = c0 + 1

            pltpu.make_async_copy(in_slice(c0), in0, si0).wait()

            @pl.when(it > 0)
            def _():
                pltpu.make_async_copy(out0, out_slice(c0), so0).wait()

            compute(in0, out0)
            pltpu.async_copy(out0, out_slice(c0), so0)

            @pl.when(it < _N_ITERS - 1)
            def _():
                pltpu.async_copy(in_slice(c0 + 2), in0, si0)

            pltpu.make_async_copy(in_slice(c1), in1, si1).wait()

            @pl.when(it > 0)
            def _():
                pltpu.make_async_copy(out1, out_slice(c1), so1).wait()

            compute(in1, out1)
            pltpu.async_copy(out1, out_slice(c1), so1)

            @pl.when(it < _N_ITERS - 1)
            def _():
                pltpu.async_copy(in_slice(c1 + 2), in1, si1)

            return ()

        lax.fori_loop(0, _N_ITERS, iter_body, ())
        pltpu.make_async_copy(out0, out_slice(_N_CHUNKS - 2), so0).wait()
        pltpu.make_async_copy(out1, out_slice(_N_CHUNKS - 1), so1).wait()

    return sc_prune


_sc_prune = _make_sc_prune()


def kernel(inputs, update_mask, apply_mask):
    flag = (update_mask != 0) & (apply_mask != 0)
    flag_v = jnp.broadcast_to(jnp.asarray(flag, jnp.int32), (16,))
    return _sc_prune(inputs, flag_v)
